# Initial kernel scaffold; baseline (speedup 1.0000x reference)
import functools

import jax
import jax.numpy as jnp
from jax import lax
from jax.experimental import pallas as pl
from jax.experimental.pallas import tpu as pltpu
from jax.experimental.pallas import tpu_sc as plsc

_NC = 2
_NS = 16
_NW = _NC * _NS
_CHUNK = 128


def _table_body(charge_ref, temp_ref, emb_ref, w1_ref, w2_ref, wp_ref, out_ref):
    B = charge_ref.shape[0]
    V, D = emb_ref.shape
    charge = charge_ref[...]
    onehot = (charge == lax.broadcasted_iota(jnp.int32, (B, V), 1)).astype(jnp.float32)
    e_charge = jnp.dot(onehot, emb_ref[...], preferred_element_type=jnp.float32)
    t = temp_ref[...]
    h = t * w1_ref[...]
    h = h * jax.nn.sigmoid(h)
    e_temp = jnp.dot(h, w2_ref[...], preferred_element_type=jnp.float32)
    z = jnp.dot(e_charge, wp_ref[:D, :], preferred_element_type=jnp.float32)
    z = z + jnp.dot(e_temp, wp_ref[D:, :], preferred_element_type=jnp.float32)
    out_ref[...] = z * jax.nn.sigmoid(z)


def _make_table(charge2d, temp2d, emb_charge, W1, W2, Wp):
    B = charge2d.shape[0]
    D = emb_charge.shape[1]
    return pl.pallas_call(
        _table_body,
        out_shape=jax.ShapeDtypeStruct((B, D), jnp.float32),
    )(charge2d, temp2d, emb_charge, W1, W2, Wp)


def _make_gather(n_pad, k_per_w, D):
    mesh = plsc.VectorSubcoreMesh(
        core_axis_name="c", subcore_axis_name="s",
        num_cores=_NC, num_subcores=_NS,
    )

    @functools.partial(
        pl.kernel,
        out_type=jax.ShapeDtypeStruct((n_pad, D), jnp.float32),
        mesh=mesh,
        scratch_types=[
            pltpu.VMEM((k_per_w, _CHUNK), jnp.int32),
            pltpu.VMEM((_CHUNK, D), jnp.float32),
            pltpu.SemaphoreType.DMA,
        ],
    )
    def gather_kernel(table_hbm, idx_hbm, out_hbm, idx_v, rows_v, sem):
        wid = lax.axis_index("s") * _NC + lax.axis_index("c")
        row0 = wid * k_per_w
        pltpu.sync_copy(idx_hbm.at[pl.ds(row0, k_per_w)], idx_v)

        @pl.loop(0, k_per_w)
        def _chunk(j):
            pltpu.async_copy(table_hbm.at[idx_v.at[j]], rows_v, sem).wait()
            pltpu.sync_copy(rows_v, out_hbm.at[pl.ds((row0 + j) * _CHUNK, _CHUNK)])

    return gather_kernel


def kernel(batch, charge, temperature, emb_charge, W1, W2, Wp):
    N = batch.shape[0]
    B = temperature.shape[0]
    D = emb_charge.shape[1]

    table = _make_table(
        charge.astype(jnp.int32).reshape(B, 1),
        temperature.reshape(B, 1),
        emb_charge, W1, W2, Wp,
    )

    k_per_w = -(-N // (_NW * _CHUNK))
    n_pad = _NW * k_per_w * _CHUNK
    idx = jnp.pad(batch.astype(jnp.int32), (0, n_pad - N)).reshape(_NW * k_per_w, _CHUNK)
    out = _make_gather(n_pad, k_per_w, D)(table, idx)
    return out[:N]


# speedup vs baseline: 3.8576x; 3.8576x over previous
import functools

import jax
import jax.numpy as jnp
from jax import lax
from jax.experimental import pallas as pl
from jax.experimental.pallas import tpu as pltpu
from jax.experimental.pallas import tpu_sc as plsc

_NC = 2
_NS = 16
_NW = _NC * _NS
_CHUNK = 128


def _table_body(charge_ref, temp_ref, emb_ref, w1_ref, w2_ref, wp_ref, out_ref):
    B = charge_ref.shape[0]
    V, D = emb_ref.shape
    charge = charge_ref[...]
    onehot = (charge == lax.broadcasted_iota(jnp.int32, (B, V), 1)).astype(jnp.float32)
    e_charge = jnp.dot(onehot, emb_ref[...], preferred_element_type=jnp.float32)
    t = temp_ref[...]
    h = t * w1_ref[...]
    h = h * jax.nn.sigmoid(h)
    e_temp = jnp.dot(h, w2_ref[...], preferred_element_type=jnp.float32)
    z = jnp.dot(e_charge, wp_ref[:D, :], preferred_element_type=jnp.float32)
    z = z + jnp.dot(e_temp, wp_ref[D:, :], preferred_element_type=jnp.float32)
    out_ref[...] = z * jax.nn.sigmoid(z)


def _make_table(charge2d, temp2d, emb_charge, W1, W2, Wp):
    B = charge2d.shape[0]
    D = emb_charge.shape[1]
    return pl.pallas_call(
        _table_body,
        out_shape=jax.ShapeDtypeStruct((B, D), jnp.float32),
    )(charge2d, temp2d, emb_charge, W1, W2, Wp)


def _make_gather(n_pad, k_per_w, D):
    mesh = plsc.VectorSubcoreMesh(
        core_axis_name="c", subcore_axis_name="s",
        num_cores=_NC, num_subcores=_NS,
    )

    @functools.partial(
        pl.kernel,
        out_type=jax.ShapeDtypeStruct((n_pad, D), jnp.float32),
        mesh=mesh,
        scratch_types=[
            pltpu.VMEM((k_per_w, _CHUNK), jnp.int32),
            pltpu.VMEM((_CHUNK, D), jnp.float32),
            pltpu.SemaphoreType.DMA,
        ],
        compiler_params=pltpu.CompilerParams(use_tc_tiling_on_sc=False),
    )
    def gather_kernel(table_hbm, idx_hbm, out_hbm, idx_v, rows_v, sem):
        wid = lax.axis_index("s") * _NC + lax.axis_index("c")
        row0 = wid * k_per_w
        pltpu.sync_copy(idx_hbm.at[wid], idx_v)

        @pl.loop(0, k_per_w)
        def _chunk(j):
            pltpu.async_copy(table_hbm.at[idx_v.at[j]], rows_v, sem).wait()
            pltpu.sync_copy(rows_v, out_hbm.at[pl.ds((row0 + j) * _CHUNK, _CHUNK)])

    return gather_kernel


def kernel(batch, charge, temperature, emb_charge, W1, W2, Wp):
    N = batch.shape[0]
    B = temperature.shape[0]
    D = emb_charge.shape[1]

    table = _make_table(
        charge.astype(jnp.int32).reshape(B, 1),
        temperature.reshape(B, 1),
        emb_charge, W1, W2, Wp,
    )

    k_per_w = -(-N // (_NW * _CHUNK))
    n_pad = _NW * k_per_w * _CHUNK
    idx = jnp.pad(batch.astype(jnp.int32), (0, n_pad - N)).reshape(_NW, k_per_w, _CHUNK)
    out = _make_gather(n_pad, k_per_w, D)(table, idx)
    return out[:N]


# speedup vs baseline: 5.2842x; 1.3698x over previous
import functools

import jax
import jax.numpy as jnp
from jax import lax
from jax.experimental import pallas as pl
from jax.experimental.pallas import tpu as pltpu
from jax.experimental.pallas import tpu_sc as plsc

_NC = 2
_NS = 16
_NW = _NC * _NS
_C = 128


def _table_body(charge_ref, temp_ref, emb_ref, w1_ref, w2_ref, wp_ref, out_ref):
    B = charge_ref.shape[0]
    V, D = emb_ref.shape
    charge = charge_ref[...]
    onehot = (charge == lax.broadcasted_iota(jnp.int32, (B, V), 1)).astype(jnp.float32)
    e_charge = jnp.dot(onehot, emb_ref[...], preferred_element_type=jnp.float32)
    t = temp_ref[...]
    h = t * w1_ref[...]
    h = h * jax.nn.sigmoid(h)
    e_temp = jnp.dot(h, w2_ref[...], preferred_element_type=jnp.float32)
    z = jnp.dot(e_charge, wp_ref[:D, :], preferred_element_type=jnp.float32)
    z = z + jnp.dot(e_temp, wp_ref[D:, :], preferred_element_type=jnp.float32)
    out_ref[...] = z * jax.nn.sigmoid(z)


def _make_table(charge2d, temp2d, emb_charge, W1, W2, Wp):
    B = charge2d.shape[0]
    D = emb_charge.shape[1]
    return pl.pallas_call(
        _table_body,
        out_shape=jax.ShapeDtypeStruct((B, D), jnp.float32),
    )(charge2d, temp2d, emb_charge, W1, W2, Wp)


def _make_gather(N, k_per_w, n_chunks, D):
    mesh = plsc.VectorSubcoreMesh(
        core_axis_name="c", subcore_axis_name="s",
        num_cores=_NC, num_subcores=_NS,
    )

    @functools.partial(
        pl.kernel,
        out_type=jax.ShapeDtypeStruct((N, D), jnp.float32),
        mesh=mesh,
        scratch_types=[
            pltpu.VMEM((k_per_w, _C), jnp.int32),
            pltpu.VMEM((_C, D), jnp.float32),
            pltpu.VMEM((_C, D), jnp.float32),
            pltpu.SemaphoreType.DMA,
            pltpu.SemaphoreType.DMA,
            pltpu.SemaphoreType.DMA,
            pltpu.SemaphoreType.DMA,
        ],
        compiler_params=pltpu.CompilerParams(use_tc_tiling_on_sc=False),
    )
    def gather_kernel(table_hbm, idx_hbm, out_hbm, idx_v, rows0, rows1,
                      sem_g0, sem_g1, sem_w0, sem_w1):
        wid = lax.axis_index("s") * _NC + lax.axis_index("c")
        c0 = wid * k_per_w
        nfull = jnp.clip(n_chunks - c0, 0, k_per_w)
        pltpu.sync_copy(idx_hbm.at[wid], idx_v)

        def out_off(j):
            return jnp.minimum((c0 + j) * _C, N - _C)

        def fire_gather(j, rows, sem):
            pltpu.async_copy(table_hbm.at[idx_v.at[j]], rows, sem)

        def fire_write(j, rows, sem):
            pltpu.async_copy(rows, out_hbm.at[pl.ds(out_off(j), _C)], sem)

        def wait_gather(rows, sem):
            pltpu.make_async_copy(out_hbm.at[pl.ds(0, _C)], rows, sem).wait()

        def wait_write(rows, sem):
            pltpu.make_async_copy(rows, out_hbm.at[pl.ds(0, _C)], sem).wait()

        @pl.when(nfull > 0)
        def _():
            fire_gather(0, rows0, sem_g0)

        @pl.loop(0, nfull)
        def _chunk(j):
            even = (j % 2) == 0

            @pl.when((j + 1 < nfull) & even)
            def _():
                @pl.when(j >= 1)
                def _():
                    wait_write(rows1, sem_w1)
                fire_gather(j + 1, rows1, sem_g1)

            @pl.when((j + 1 < nfull) & jnp.logical_not(even))
            def _():
                wait_write(rows0, sem_w0)
                fire_gather(j + 1, rows0, sem_g0)

            @pl.when(even)
            def _():
                wait_gather(rows0, sem_g0)
                fire_write(j, rows0, sem_w0)

            @pl.when(jnp.logical_not(even))
            def _():
                wait_gather(rows1, sem_g1)
                fire_write(j, rows1, sem_w1)

        @pl.when(nfull >= 1)
        def _():
            wait_write(rows0, sem_w0)

        @pl.when(nfull >= 2)
        def _():
            wait_write(rows1, sem_w1)

    return gather_kernel


def kernel(batch, charge, temperature, emb_charge, W1, W2, Wp):
    N = batch.shape[0]
    B = temperature.shape[0]
    D = emb_charge.shape[1]

    table = _make_table(
        charge.astype(jnp.int32).reshape(B, 1),
        temperature.reshape(B, 1),
        emb_charge, W1, W2, Wp,
    )

    n_chunks = -(-N // _C)
    k_per_w = -(-n_chunks // _NW)
    batch = batch.astype(jnp.int32)
    idx = jnp.pad(batch, (0, _NW * k_per_w * _C - N)).reshape(_NW, k_per_w, _C)
    last = n_chunks - 1
    idx = idx.at[last // k_per_w, last % k_per_w].set(batch[N - _C:])
    return _make_gather(N, k_per_w, n_chunks, D)(table, idx)
